# baseline (speedup 1.0000x reference)
import functools

import jax
import jax.numpy as jnp
from jax import lax
from jax.experimental import pallas as pl
from jax.experimental.pallas import tpu as pltpu
from jax.experimental.pallas import tpu_sc as plsc

NUM_PIDS = 1024
SEG = 8192
LEN = 128
TOTAL = SEG * LEN

R_GAS = 0.0083144598
KELVIN = 273.15
TROOM = 25.0 + KELVIN
DHA_RD = 46.39
DHD_VCMAX = 200.0
DHD_JMAX = 200.0
DHD_TPU = 201.8

NC = 2
NS = 16
LANES = 16
NW = NC * NS
SEG_PER_W = SEG // NW


NCOEF = 12
NCOEF_PAD = 16
BLK_PER_W = SEG_PER_W // LEN
CHUNK_C = NCOEF_PAD * LEN
LOG2E = 1.4426950408889634


def _sc_gather_body(pids_hbm, dV_hbm, dJ_hbm, dT_hbm, tV_hbm, tJ_hbm, tT_hbm,
                    coef_hbm,
                    pids_v, pids8_v, dVv, dJv, dTv, tVv, tJv, tTv,
                    buf, sem):
    wid = lax.axis_index("s") * NC + lax.axis_index("c")
    base = wid * SEG_PER_W
    descs = [
        pltpu.async_copy(pids_hbm.at[pl.ds(base, SEG_PER_W)], pids_v, sem),
        pltpu.async_copy(pids_hbm.at[pl.ds(0, LANES)], pids8_v, sem),
        pltpu.async_copy(dV_hbm, dVv, sem),
        pltpu.async_copy(dJ_hbm, dJv, sem),
        pltpu.async_copy(dT_hbm, dTv, sem),
        pltpu.async_copy(tV_hbm, tVv, sem),
        pltpu.async_copy(tJ_hbm, tJv, sem),
        pltpu.async_copy(tT_hbm, tTv, sem),
    ]
    for d in descs:
        d.wait()
    c_rk2 = jnp.float32(LOG2E / (R_GAS * TROOM))
    c_r2 = jnp.float32(LOG2E / R_GAS)
    rec_troom = jnp.float32(1.0 / TROOM)
    chans = ((dVv, tVv, jnp.float32(DHD_VCMAX), jnp.float32(DHD_VCMAX / R_GAS)),
             (dJv, tJv, jnp.float32(DHD_JMAX), jnp.float32(DHD_JMAX / R_GAS)),
             (dTv, tTv, jnp.float32(DHD_TPU), jnp.float32(DHD_TPU / R_GAS)))
    for i in range(SEG_PER_W // LANES):
        p = pids_v[pl.ds(i * LANES, LANES)]
        q = plsc.load_gather(pids8_v, [jnp.right_shift(p, 7)])
        off = (i // 8) * CHUNK_C + (i % 8) * LANES
        for ch, (dv, tv, dhd, dhd_r) in enumerate(chans):
            a1 = plsc.load_gather(dv, [p])
            a2 = plsc.load_gather(dv, [q])
            tp = plsc.load_gather(tv, [p])
            g = a1 / (dhd - a1)
            rtp = 1.0 / tp
            buf[pl.ds(off + (4 * ch + 0) * LEN, LANES)] = a2 * c_rk2
            buf[pl.ds(off + (4 * ch + 1) * LEN, LANES)] = a2 * c_r2
            buf[pl.ds(off + (4 * ch + 2) * LEN, LANES)] = \
                1.0 + g * jnp.exp(dhd_r * (rtp - rec_troom))
            buf[pl.ds(off + (4 * ch + 3) * LEN, LANES)] = \
                g * jnp.exp(dhd_r * rtp)
    pltpu.sync_copy(
        buf, coef_hbm.at[pl.ds(wid * BLK_PER_W * CHUNK_C,
                               BLK_PER_W * CHUNK_C)])


def _sc_gather(pids, dV, dJ, dT, tV, tJ, tT):
    mesh = plsc.VectorSubcoreMesh(core_axis_name="c", subcore_axis_name="s",
                                  num_cores=NC, num_subcores=NS)
    return pl.kernel(
        _sc_gather_body,
        out_type=jax.ShapeDtypeStruct((SEG // LEN * CHUNK_C,), jnp.float32),
        mesh=mesh,
        compiler_params=pltpu.CompilerParams(needs_layout_passes=False),
        scratch_types=[
            pltpu.VMEM((SEG_PER_W,), jnp.int32),
            pltpu.VMEM((LANES,), jnp.int32),
        ] + [pltpu.VMEM((NUM_PIDS,), jnp.float32) for _ in range(6)]
          + [pltpu.VMEM((BLK_PER_W * CHUNK_C,), jnp.float32),
             pltpu.SemaphoreType.DMA],
    )(pids, dV, dJ, dT, tV, tJ, tT)


NB = 4
BS = NB * LEN


def _tc_body(tleaf, vc25, jm25, tp25, rd25, coef, out_ref):
    sub = LEN * LEN
    d_vj2 = jnp.float32(-LOG2E * DHD_VCMAX / R_GAS)
    d_t2 = jnp.float32(-LOG2E * DHD_TPU / R_GAS)
    ard = jnp.float32(LOG2E * DHA_RD / (R_GAS * TROOM))
    brd = jnp.float32(LOG2E * DHA_RD / R_GAS)

    D = coef[...].reshape(NB * NCOEF_PAD, LEN)
    Dpad = jnp.concatenate(
        [D, jnp.zeros((LEN - NB * NCOEF_PAD, LEN), jnp.float32)], axis=0)
    T = jnp.transpose(Dpad)
    t2 = tleaf[...].reshape(BS, LEN)
    k25s = (vc25, jm25, tp25)
    for b in range(NB):
        r = 1.0 / t2[b * LEN:(b + 1) * LEN, :]
        e_vj = jnp.exp2(d_vj2 * r)
        e_t = jnp.exp2(d_t2 * r)
        es = (e_vj, e_vj, e_t)
        cb = T[:, b * NCOEF_PAD:(b + 1) * NCOEF_PAD]
        for ch in range(3):
            A = cb[:, 4 * ch + 0:4 * ch + 1]
            B = cb[:, 4 * ch + 1:4 * ch + 2]
            numc = cb[:, 4 * ch + 2:4 * ch + 3]
            G = cb[:, 4 * ch + 3:4 * ch + 4]
            k25 = k25s[ch][...].reshape(BS, LEN)[b * LEN:(b + 1) * LEN, :]
            res = k25 * numc * jnp.exp2(A - B * r) / (1.0 + G * es[ch])
            out_ref[ch, pl.ds(b * sub, sub)] = res.reshape(sub)
        rd = rd25[...].reshape(BS, LEN)[b * LEN:(b + 1) * LEN, :]
        out_ref[3, pl.ds(b * sub, sub)] = (
            rd * jnp.exp2(ard - brd * r)).reshape(sub)


def kernel(Tleaf, Vcmax25, Jmax25, TPU25, Rd25, dHa_Vcmax, dHa_Jmax, dHa_TPU,
           Topt_Vcmax, Topt_Jmax, Topt_TPU, PIDs, lengths):
    del lengths
    coefs = _sc_gather(PIDs, dHa_Vcmax, dHa_Jmax, dHa_TPU,
                       Topt_Vcmax, Topt_Jmax, Topt_TPU)
    elems = (Tleaf, Vcmax25, Jmax25, TPU25, Rd25)

    chunk = BS * LEN
    eblk = pl.BlockSpec((chunk,), lambda i: (i,))
    cblk = pl.BlockSpec((BS // LEN * CHUNK_C,), lambda i: (i,))
    return pl.pallas_call(
        _tc_body,
        grid=(SEG // BS,),
        in_specs=[eblk] * 5 + [cblk],
        out_specs=pl.BlockSpec((4, chunk), lambda i: (0, i)),
        out_shape=jax.ShapeDtypeStruct((4, TOTAL), jnp.float32),
    )(*elems, coefs)


# speedup vs baseline: 1.0906x; 1.0906x over previous
import functools

import jax
import jax.numpy as jnp
from jax import lax
from jax.experimental import pallas as pl
from jax.experimental.pallas import tpu as pltpu
from jax.experimental.pallas import tpu_sc as plsc

NUM_PIDS = 1024
SEG = 8192
LEN = 128
TOTAL = SEG * LEN

R_GAS = 0.0083144598
KELVIN = 273.15
TROOM = 25.0 + KELVIN
DHA_RD = 46.39
DHD_VCMAX = 200.0
DHD_JMAX = 200.0
DHD_TPU = 201.8

NC = 2
NS = 16
LANES = 16
NW = NC * NS
SEG_PER_W = SEG // NW


NCOEF = 12
NCOEF_PAD = 16
BLK_PER_W = SEG_PER_W // LEN
CHUNK_C = NCOEF_PAD * LEN
LOG2E = 1.4426950408889634


def _sc_gather_body(pids_hbm, dV_hbm, dJ_hbm, dT_hbm, tV_hbm, tJ_hbm, tT_hbm,
                    coef_hbm,
                    pids_v, pids8_v, dVv, dJv, dTv, tVv, tJv, tTv,
                    buf, sem):
    wid = lax.axis_index("s") * NC + lax.axis_index("c")
    base = wid * SEG_PER_W
    descs = [
        pltpu.async_copy(pids_hbm.at[pl.ds(base, SEG_PER_W)], pids_v, sem),
        pltpu.async_copy(pids_hbm.at[pl.ds(0, LANES)], pids8_v, sem),
        pltpu.async_copy(dV_hbm, dVv, sem),
        pltpu.async_copy(dJ_hbm, dJv, sem),
        pltpu.async_copy(dT_hbm, dTv, sem),
        pltpu.async_copy(tV_hbm, tVv, sem),
        pltpu.async_copy(tJ_hbm, tJv, sem),
        pltpu.async_copy(tT_hbm, tTv, sem),
    ]
    for d in descs:
        d.wait()
    c_rk2 = jnp.float32(LOG2E / (R_GAS * TROOM))
    c_r2 = jnp.float32(LOG2E / R_GAS)
    rec_troom = jnp.float32(1.0 / TROOM)
    chans = ((dVv, tVv, jnp.float32(DHD_VCMAX), jnp.float32(DHD_VCMAX / R_GAS)),
             (dJv, tJv, jnp.float32(DHD_JMAX), jnp.float32(DHD_JMAX / R_GAS)),
             (dTv, tTv, jnp.float32(DHD_TPU), jnp.float32(DHD_TPU / R_GAS)))
    for i in range(SEG_PER_W // LANES):
        p = pids_v[pl.ds(i * LANES, LANES)]
        q = plsc.load_gather(pids8_v, [jnp.right_shift(p, 7)])
        off = (i // 8) * CHUNK_C + (i % 8) * LANES
        for ch, (dv, tv, dhd, dhd_r) in enumerate(chans):
            a1 = plsc.load_gather(dv, [p])
            a2 = plsc.load_gather(dv, [q])
            tp = plsc.load_gather(tv, [p])
            g = a1 / (dhd - a1)
            rtp = 1.0 / tp
            buf[pl.ds(off + (4 * ch + 0) * LEN, LANES)] = a2 * c_rk2
            buf[pl.ds(off + (4 * ch + 1) * LEN, LANES)] = a2 * c_r2
            buf[pl.ds(off + (4 * ch + 2) * LEN, LANES)] = \
                1.0 + g * jnp.exp(dhd_r * (rtp - rec_troom))
            buf[pl.ds(off + (4 * ch + 3) * LEN, LANES)] = \
                g * jnp.exp(dhd_r * rtp)
    pltpu.sync_copy(
        buf, coef_hbm.at[pl.ds(wid * BLK_PER_W * CHUNK_C,
                               BLK_PER_W * CHUNK_C)])


def _sc_gather(pids, dV, dJ, dT, tV, tJ, tT):
    mesh = plsc.VectorSubcoreMesh(core_axis_name="c", subcore_axis_name="s",
                                  num_cores=NC, num_subcores=NS)
    return pl.kernel(
        _sc_gather_body,
        out_type=jax.ShapeDtypeStruct((SEG // LEN * CHUNK_C,), jnp.float32),
        mesh=mesh,
        compiler_params=pltpu.CompilerParams(needs_layout_passes=False),
        scratch_types=[
            pltpu.VMEM((SEG_PER_W,), jnp.int32),
            pltpu.VMEM((LANES,), jnp.int32),
        ] + [pltpu.VMEM((NUM_PIDS,), jnp.float32) for _ in range(6)]
          + [pltpu.VMEM((BLK_PER_W * CHUNK_C,), jnp.float32),
             pltpu.SemaphoreType.DMA],
    )(pids, dV, dJ, dT, tV, tJ, tT)


NB = 8
BS = NB * LEN


def _tc_body(tleaf, vc25, jm25, tp25, rd25, coef, out_ref):
    sub = LEN * LEN
    d_vj2 = jnp.float32(-LOG2E * DHD_VCMAX / R_GAS)
    d_t2 = jnp.float32(-LOG2E * DHD_TPU / R_GAS)
    ard = jnp.float32(LOG2E * DHA_RD / (R_GAS * TROOM))
    brd = jnp.float32(LOG2E * DHA_RD / R_GAS)

    D = coef[...].reshape(NB * NCOEF_PAD, LEN)
    if NB * NCOEF_PAD < LEN:
        D = jnp.concatenate(
            [D, jnp.zeros((LEN - NB * NCOEF_PAD, LEN), jnp.float32)], axis=0)
    Dpad = D
    T = jnp.transpose(Dpad)
    t2 = tleaf[...].reshape(BS, LEN)
    k25s = (vc25, jm25, tp25)
    for b in range(NB):
        r = 1.0 / t2[b * LEN:(b + 1) * LEN, :]
        e_vj = jnp.exp2(d_vj2 * r)
        e_t = jnp.exp2(d_t2 * r)
        es = (e_vj, e_vj, e_t)
        cb = T[:, b * NCOEF_PAD:(b + 1) * NCOEF_PAD]
        for ch in range(3):
            A = cb[:, 4 * ch + 0:4 * ch + 1]
            B = cb[:, 4 * ch + 1:4 * ch + 2]
            numc = cb[:, 4 * ch + 2:4 * ch + 3]
            G = cb[:, 4 * ch + 3:4 * ch + 4]
            k25 = k25s[ch][...].reshape(BS, LEN)[b * LEN:(b + 1) * LEN, :]
            res = k25 * numc * jnp.exp2(A - B * r) / (1.0 + G * es[ch])
            out_ref[ch, pl.ds(b * sub, sub)] = res.reshape(sub)
        rd = rd25[...].reshape(BS, LEN)[b * LEN:(b + 1) * LEN, :]
        out_ref[3, pl.ds(b * sub, sub)] = (
            rd * jnp.exp2(ard - brd * r)).reshape(sub)


def kernel(Tleaf, Vcmax25, Jmax25, TPU25, Rd25, dHa_Vcmax, dHa_Jmax, dHa_TPU,
           Topt_Vcmax, Topt_Jmax, Topt_TPU, PIDs, lengths):
    del lengths
    coefs = _sc_gather(PIDs, dHa_Vcmax, dHa_Jmax, dHa_TPU,
                       Topt_Vcmax, Topt_Jmax, Topt_TPU)
    elems = (Tleaf, Vcmax25, Jmax25, TPU25, Rd25)

    chunk = BS * LEN
    eblk = pl.BlockSpec((chunk,), lambda i: (i,))
    cblk = pl.BlockSpec((BS // LEN * CHUNK_C,), lambda i: (i,))
    return pl.pallas_call(
        _tc_body,
        grid=(SEG // BS,),
        in_specs=[eblk] * 5 + [cblk],
        out_specs=pl.BlockSpec((4, chunk), lambda i: (0, i)),
        out_shape=jax.ShapeDtypeStruct((4, TOTAL), jnp.float32),
    )(*elems, coefs)


# speedup vs baseline: 1.1172x; 1.0244x over previous
import functools

import jax
import jax.numpy as jnp
from jax import lax
from jax.experimental import pallas as pl
from jax.experimental.pallas import tpu as pltpu
from jax.experimental.pallas import tpu_sc as plsc

NUM_PIDS = 1024
SEG = 8192
LEN = 128
TOTAL = SEG * LEN

R_GAS = 0.0083144598
KELVIN = 273.15
TROOM = 25.0 + KELVIN
DHA_RD = 46.39
DHD_VCMAX = 200.0
DHD_JMAX = 200.0
DHD_TPU = 201.8

NC = 2
NS = 16
LANES = 16
NW = NC * NS
SEG_PER_W = SEG // NW


NCOEF = 12
NCOEF_PAD = 16
BLK_PER_W = SEG_PER_W // LEN
CHUNK_C = NCOEF_PAD * LEN
LOG2E = 1.4426950408889634


def _sc_gather_body(pids_hbm, dV_hbm, dJ_hbm, dT_hbm, tV_hbm, tJ_hbm, tT_hbm,
                    coef_hbm,
                    pids_v, pids8_v, dVv, dJv, dTv, tVv, tJv, tTv,
                    buf, sem):
    wid = lax.axis_index("s") * NC + lax.axis_index("c")
    base = wid * SEG_PER_W
    descs = [
        pltpu.async_copy(pids_hbm.at[pl.ds(base, SEG_PER_W)], pids_v, sem),
        pltpu.async_copy(pids_hbm.at[pl.ds(0, LANES)], pids8_v, sem),
        pltpu.async_copy(dV_hbm, dVv, sem),
        pltpu.async_copy(dJ_hbm, dJv, sem),
        pltpu.async_copy(dT_hbm, dTv, sem),
        pltpu.async_copy(tV_hbm, tVv, sem),
        pltpu.async_copy(tJ_hbm, tJv, sem),
        pltpu.async_copy(tT_hbm, tTv, sem),
    ]
    for d in descs:
        d.wait()
    c_rk2 = jnp.float32(LOG2E / (R_GAS * TROOM))
    c_r2 = jnp.float32(LOG2E / R_GAS)
    rec_troom = jnp.float32(1.0 / TROOM)
    chans = ((dVv, tVv, jnp.float32(DHD_VCMAX), jnp.float32(DHD_VCMAX / R_GAS)),
             (dJv, tJv, jnp.float32(DHD_JMAX), jnp.float32(DHD_JMAX / R_GAS)),
             (dTv, tTv, jnp.float32(DHD_TPU), jnp.float32(DHD_TPU / R_GAS)))
    for i in range(SEG_PER_W // LANES):
        p = pids_v[pl.ds(i * LANES, LANES)]
        q = plsc.load_gather(pids8_v, [jnp.right_shift(p, 7)])
        off = (i // 8) * CHUNK_C + (i % 8) * LANES
        for ch, (dv, tv, dhd, dhd_r) in enumerate(chans):
            a1 = plsc.load_gather(dv, [p])
            a2 = plsc.load_gather(dv, [q])
            tp = plsc.load_gather(tv, [p])
            g = a1 / (dhd - a1)
            rtp = 1.0 / tp
            buf[pl.ds(off + (4 * ch + 0) * LEN, LANES)] = a2 * c_rk2
            buf[pl.ds(off + (4 * ch + 1) * LEN, LANES)] = a2 * c_r2
            buf[pl.ds(off + (4 * ch + 2) * LEN, LANES)] = \
                1.0 + g * jnp.exp(dhd_r * (rtp - rec_troom))
            buf[pl.ds(off + (4 * ch + 3) * LEN, LANES)] = \
                g * jnp.exp(dhd_r * rtp)
    pltpu.sync_copy(
        buf, coef_hbm.at[pl.ds(wid * BLK_PER_W * CHUNK_C,
                               BLK_PER_W * CHUNK_C)])


def _sc_gather(pids, dV, dJ, dT, tV, tJ, tT):
    mesh = plsc.VectorSubcoreMesh(core_axis_name="c", subcore_axis_name="s",
                                  num_cores=NC, num_subcores=NS)
    return pl.kernel(
        _sc_gather_body,
        out_type=jax.ShapeDtypeStruct((SEG // LEN * CHUNK_C,), jnp.float32),
        mesh=mesh,
        compiler_params=pltpu.CompilerParams(needs_layout_passes=False),
        scratch_types=[
            pltpu.VMEM((SEG_PER_W,), jnp.int32),
            pltpu.VMEM((LANES,), jnp.int32),
        ] + [pltpu.VMEM((NUM_PIDS,), jnp.float32) for _ in range(6)]
          + [pltpu.VMEM((BLK_PER_W * CHUNK_C,), jnp.float32),
             pltpu.SemaphoreType.DMA],
    )(pids, dV, dJ, dT, tV, tJ, tT)


NB = 16
BS = NB * LEN


def _tc_body(tleaf, vc25, jm25, tp25, rd25, coef, out_ref):
    sub = LEN * LEN
    d_vj2 = jnp.float32(-LOG2E * DHD_VCMAX / R_GAS)
    d_t2 = jnp.float32(-LOG2E * DHD_TPU / R_GAS)
    ard = jnp.float32(LOG2E * DHA_RD / (R_GAS * TROOM))
    brd = jnp.float32(LOG2E * DHA_RD / R_GAS)

    D = coef[...].reshape(NB * NCOEF_PAD, LEN)
    if NB * NCOEF_PAD < LEN:
        D = jnp.concatenate(
            [D, jnp.zeros((LEN - NB * NCOEF_PAD, LEN), jnp.float32)], axis=0)
    Dpad = D
    T = jnp.transpose(Dpad)
    t2 = tleaf[...].reshape(BS, LEN)
    k25s = (vc25, jm25, tp25)
    for b in range(NB):
        r = 1.0 / t2[b * LEN:(b + 1) * LEN, :]
        e_vj = jnp.exp2(d_vj2 * r)
        e_t = jnp.exp2(d_t2 * r)
        es = (e_vj, e_vj, e_t)
        cb = T[:, b * NCOEF_PAD:(b + 1) * NCOEF_PAD]
        for ch in range(3):
            A = cb[:, 4 * ch + 0:4 * ch + 1]
            B = cb[:, 4 * ch + 1:4 * ch + 2]
            numc = cb[:, 4 * ch + 2:4 * ch + 3]
            G = cb[:, 4 * ch + 3:4 * ch + 4]
            k25 = k25s[ch][...].reshape(BS, LEN)[b * LEN:(b + 1) * LEN, :]
            res = k25 * numc * jnp.exp2(A - B * r) / (1.0 + G * es[ch])
            out_ref[ch, pl.ds(b * sub, sub)] = res.reshape(sub)
        rd = rd25[...].reshape(BS, LEN)[b * LEN:(b + 1) * LEN, :]
        out_ref[3, pl.ds(b * sub, sub)] = (
            rd * jnp.exp2(ard - brd * r)).reshape(sub)


def kernel(Tleaf, Vcmax25, Jmax25, TPU25, Rd25, dHa_Vcmax, dHa_Jmax, dHa_TPU,
           Topt_Vcmax, Topt_Jmax, Topt_TPU, PIDs, lengths):
    del lengths
    coefs = _sc_gather(PIDs, dHa_Vcmax, dHa_Jmax, dHa_TPU,
                       Topt_Vcmax, Topt_Jmax, Topt_TPU)
    elems = (Tleaf, Vcmax25, Jmax25, TPU25, Rd25)

    chunk = BS * LEN
    eblk = pl.BlockSpec((chunk,), lambda i: (i,))
    cblk = pl.BlockSpec((BS // LEN * CHUNK_C,), lambda i: (i,))
    return pl.pallas_call(
        _tc_body,
        grid=(SEG // BS,),
        in_specs=[eblk] * 5 + [cblk],
        out_specs=pl.BlockSpec((4, chunk), lambda i: (0, i)),
        out_shape=jax.ShapeDtypeStruct((4, TOTAL), jnp.float32),
    )(*elems, coefs)


# speedup vs baseline: 1.1603x; 1.0386x over previous
import functools

import jax
import jax.numpy as jnp
from jax import lax
from jax.experimental import pallas as pl
from jax.experimental.pallas import tpu as pltpu
from jax.experimental.pallas import tpu_sc as plsc

NUM_PIDS = 1024
SEG = 8192
LEN = 128
TOTAL = SEG * LEN

R_GAS = 0.0083144598
KELVIN = 273.15
TROOM = 25.0 + KELVIN
DHA_RD = 46.39
DHD_VCMAX = 200.0
DHD_JMAX = 200.0
DHD_TPU = 201.8

NC = 2
NS = 16
LANES = 16
NW = NC * NS
SEG_PER_W = SEG // NW


NCOEF = 9
NCOEF_PAD = 16
BLK_PER_W = SEG_PER_W // LEN
CHUNK_C = NCOEF_PAD * LEN
LOG2E = 1.4426950408889634


def _sc_gather_body(pids_hbm, dV_hbm, dJ_hbm, dT_hbm, tV_hbm, tJ_hbm, tT_hbm,
                    coef_hbm,
                    pids_v, pids8_v, dVv, dJv, dTv, tVv, tJv, tTv,
                    buf, sem):
    wid = lax.axis_index("s") * NC + lax.axis_index("c")
    base = wid * SEG_PER_W
    descs = [
        pltpu.async_copy(pids_hbm.at[pl.ds(base, SEG_PER_W)], pids_v, sem),
        pltpu.async_copy(pids_hbm.at[pl.ds(0, LANES)], pids8_v, sem),
        pltpu.async_copy(dV_hbm, dVv, sem),
        pltpu.async_copy(dJ_hbm, dJv, sem),
        pltpu.async_copy(dT_hbm, dTv, sem),
        pltpu.async_copy(tV_hbm, tVv, sem),
        pltpu.async_copy(tJ_hbm, tJv, sem),
        pltpu.async_copy(tT_hbm, tTv, sem),
    ]
    for d in descs:
        d.wait()
    c_r2 = jnp.float32(LOG2E / R_GAS)
    rec_troom = jnp.float32(1.0 / TROOM)
    chans = ((dVv, tVv, jnp.float32(DHD_VCMAX), jnp.float32(DHD_VCMAX / R_GAS)),
             (dJv, tJv, jnp.float32(DHD_JMAX), jnp.float32(DHD_JMAX / R_GAS)),
             (dTv, tTv, jnp.float32(DHD_TPU), jnp.float32(DHD_TPU / R_GAS)))
    for i in range(SEG_PER_W // LANES):
        p = pids_v[pl.ds(i * LANES, LANES)]
        q = plsc.load_gather(pids8_v, [jnp.right_shift(p, 7)])
        off = (i // 8) * CHUNK_C + (i % 8) * LANES
        for ch, (dv, tv, dhd, dhd_r) in enumerate(chans):
            a1 = plsc.load_gather(dv, [p])
            a2 = plsc.load_gather(dv, [q])
            tp = plsc.load_gather(tv, [p])
            g = a1 / (dhd - a1)
            rtp = 1.0 / tp
            buf[pl.ds(off + (3 * ch + 0) * LEN, LANES)] = a2 * c_r2
            buf[pl.ds(off + (3 * ch + 1) * LEN, LANES)] = \
                1.0 + g * jnp.exp(dhd_r * (rtp - rec_troom))
            buf[pl.ds(off + (3 * ch + 2) * LEN, LANES)] = \
                g * jnp.exp(dhd_r * rtp)
    pltpu.sync_copy(
        buf, coef_hbm.at[pl.ds(wid * BLK_PER_W * CHUNK_C,
                               BLK_PER_W * CHUNK_C)])


def _sc_gather(pids, dV, dJ, dT, tV, tJ, tT):
    mesh = plsc.VectorSubcoreMesh(core_axis_name="c", subcore_axis_name="s",
                                  num_cores=NC, num_subcores=NS)
    return pl.kernel(
        _sc_gather_body,
        out_type=jax.ShapeDtypeStruct((SEG // LEN * CHUNK_C,), jnp.float32),
        mesh=mesh,
        compiler_params=pltpu.CompilerParams(needs_layout_passes=False),
        scratch_types=[
            pltpu.VMEM((SEG_PER_W,), jnp.int32),
            pltpu.VMEM((LANES,), jnp.int32),
        ] + [pltpu.VMEM((NUM_PIDS,), jnp.float32) for _ in range(6)]
          + [pltpu.VMEM((BLK_PER_W * CHUNK_C,), jnp.float32),
             pltpu.SemaphoreType.DMA],
    )(pids, dV, dJ, dT, tV, tJ, tT)


NB = 32
BS = NB * LEN


def _tc_body(tleaf, vc25, jm25, tp25, rd25, coef, out_ref):
    sub = LEN * LEN
    d_vj2 = jnp.float32(-LOG2E * DHD_VCMAX / R_GAS)
    d_t2 = jnp.float32(-LOG2E * DHD_TPU / R_GAS)
    ard = jnp.float32(LOG2E * DHA_RD / (R_GAS * TROOM))
    brd = jnp.float32(LOG2E * DHA_RD / R_GAS)

    D = coef[...].reshape(NB * NCOEF_PAD, LEN)
    if NB * NCOEF_PAD < LEN:
        D = jnp.concatenate(
            [D, jnp.zeros((LEN - NB * NCOEF_PAD, LEN), jnp.float32)], axis=0)
    Dpad = D
    T = jnp.transpose(Dpad)
    t2 = tleaf[...].reshape(BS, LEN)
    k25s = (vc25, jm25, tp25)
    rec_troom = jnp.float32(1.0 / TROOM)
    for b in range(NB):
        r = 1.0 / t2[b * LEN:(b + 1) * LEN, :]
        dr = rec_troom - r
        e_vj = jnp.exp2(d_vj2 * r)
        e_t = jnp.exp2(d_t2 * r)
        es = (e_vj, e_vj, e_t)
        cb = T[:, b * NCOEF_PAD:(b + 1) * NCOEF_PAD]
        for ch in range(3):
            B = cb[:, 3 * ch + 0:3 * ch + 1]
            numc = cb[:, 3 * ch + 1:3 * ch + 2]
            G = cb[:, 3 * ch + 2:3 * ch + 3]
            k25 = k25s[ch][...].reshape(BS, LEN)[b * LEN:(b + 1) * LEN, :]
            res = k25 * numc * jnp.exp2(B * dr) / (1.0 + G * es[ch])
            out_ref[ch, pl.ds(b * sub, sub)] = res.reshape(sub)
        rd = rd25[...].reshape(BS, LEN)[b * LEN:(b + 1) * LEN, :]
        out_ref[3, pl.ds(b * sub, sub)] = (
            rd * jnp.exp2(ard - brd * r)).reshape(sub)


def kernel(Tleaf, Vcmax25, Jmax25, TPU25, Rd25, dHa_Vcmax, dHa_Jmax, dHa_TPU,
           Topt_Vcmax, Topt_Jmax, Topt_TPU, PIDs, lengths):
    del lengths
    coefs = _sc_gather(PIDs, dHa_Vcmax, dHa_Jmax, dHa_TPU,
                       Topt_Vcmax, Topt_Jmax, Topt_TPU)
    elems = (Tleaf, Vcmax25, Jmax25, TPU25, Rd25)

    chunk = BS * LEN
    eblk = pl.BlockSpec((chunk,), lambda i: (i,))
    cblk = pl.BlockSpec((BS // LEN * CHUNK_C,), lambda i: (i,))
    return pl.pallas_call(
        _tc_body,
        grid=(SEG // BS,),
        in_specs=[eblk] * 5 + [cblk],
        out_specs=pl.BlockSpec((4, chunk), lambda i: (0, i)),
        out_shape=jax.ShapeDtypeStruct((4, TOTAL), jnp.float32),
    )(*elems, coefs)
